# baseline (speedup 1.0000x reference)
import functools

import numpy as np
import jax
import jax.numpy as jnp
from jax import lax
from jax.experimental import pallas as pl
from jax.experimental.pallas import tpu as pltpu
from jax.experimental.pallas import tpu_sc as plsc

B = 64
T = 1024
C = 128
MULT = 4
HIDDEN = 512
MAX_SPIKES = 256
N_POS = 1024
SCALE = float(np.sqrt(HIDDEN))
LN_EPS = 1e-5

P = B * T
NC, NS, L = 2, 16, 16
NW = NC * NS
PPW = P // NW
CH = 64
NCH = PPW // CH
NGRP = HIDDEN // (2 * L)
HIDP = HIDDEN // 2


def _prep_tables(pos, g, b, emb):
    def body(pos_ref, g_ref, b_ref, emb_ref, lnp_ref, sct_ref):
        x = pos_ref[...]
        mu = jnp.mean(x, axis=-1, keepdims=True)
        var = jnp.mean(jnp.square(x - mu), axis=-1, keepdims=True)
        lnp_ref[...] = (x - mu) / jnp.sqrt(var + LN_EPS) * g_ref[...] + b_ref[...]
        sct_ref[...] = emb_ref[...] * SCALE
    return pl.pallas_call(
        body,
        out_shape=(
            jax.ShapeDtypeStruct((N_POS, HIDDEN), jnp.float32),
            jax.ShapeDtypeStruct((MAX_SPIKES, MULT), jnp.float32),
        ),
    )(pos, g.reshape(1, HIDDEN), b.reshape(1, HIDDEN), emb)


def _pack_bf16_pairs(x):
    n, w = x.shape
    bits = lax.bitcast_convert_type(x.astype(jnp.bfloat16), jnp.uint16)
    bits = bits.astype(jnp.uint32)
    packed = bits[:, 0::2] | (bits[:, 1::2] << 16)
    return lax.bitcast_convert_type(packed, jnp.int32)


def _make_sc_kernel():
    mesh = plsc.VectorSubcoreMesh(core_axis_name="c", subcore_axis_name="s")

    @functools.partial(
        pl.kernel,
        mesh=mesh,
        out_type=jax.ShapeDtypeStruct((P, HIDDEN), jnp.float32),
        compiler_params=pltpu.CompilerParams(needs_layout_passes=False),
        scratch_types=[
            pltpu.VMEM((MAX_SPIKES * 2,), jnp.int32),
            pltpu.VMEM((CH * C,), jnp.int32),
            pltpu.VMEM((CH * C,), jnp.int32),
            pltpu.VMEM((CH,), jnp.int32),
            pltpu.VMEM((CH,), jnp.int32),
            pltpu.VMEM((CH, HIDP), jnp.int32),
            pltpu.VMEM((CH, HIDP), jnp.int32),
            pltpu.VMEM((CH, HIDDEN), jnp.float32),
            pltpu.VMEM((CH, HIDDEN), jnp.float32),
            pltpu.SemaphoreType.DMA,
            pltpu.SemaphoreType.DMA,
            pltpu.SemaphoreType.DMA,
            pltpu.SemaphoreType.DMA,
            pltpu.SemaphoreType.DMA,
            pltpu.SemaphoreType.DMA,
        ],
    )
    def sc_kernel(spk_hbm, st_hbm, lnp_hbm, sct_hbm, out_hbm,
                  sct_v, spk_v0, spk_v1, st_v0, st_v1, pos_v0, pos_v1,
                  out_v0, out_v1,
                  sem_i0, sem_i1, sem_g0, sem_g1, sem_o0, sem_o1):
        spk_v = (spk_v0, spk_v1)
        st_v = (st_v0, st_v1)
        pos_v = (pos_v0, pos_v1)
        out_v = (out_v0, out_v1)
        sem_i = (sem_i0, sem_i1)
        sem_g = (sem_g0, sem_g1)
        sem_o = (sem_o0, sem_o1)

        wid = lax.axis_index("s") * NC + lax.axis_index("c")
        base0 = wid * PPW

        pltpu.sync_copy(sct_hbm, sct_v)

        lanes = lax.iota(jnp.int32, L)
        rep2 = lax.shift_right_logical(lanes, 1)
        jpat = jnp.bitwise_and(lanes, 1)
        cole = lax.shift_left(lanes, 1)
        colo = jnp.bitwise_or(cole, 1)
        himask = jnp.full((L,), -65536, dtype=jnp.int32)

        def in_copy(ci, bi):
            base = base0 + ci * CH
            return (
                pltpu.make_async_copy(
                    spk_hbm.at[pl.ds(base * C, CH * C)], spk_v[bi], sem_i[bi]),
                pltpu.make_async_copy(
                    st_hbm.at[pl.ds(base, CH)], st_v[bi], sem_i[bi]),
            )

        def gather_copy(bi):
            return pltpu.make_async_copy(lnp_hbm.at[st_v[bi]], pos_v[bi], sem_g[bi])

        def out_copy(ci, bi):
            base = base0 + ci * CH
            return pltpu.make_async_copy(
                out_v[bi], out_hbm.at[pl.ds(base, CH)], sem_o[bi])

        def compute(bi):
            GG = 2

            @plsc.parallel_loop(0, CH, unroll=2)
            def pair_body(p):
                zero16 = jnp.bitwise_and(lanes, 0)
                pbase = jnp.full((L,), p * C, dtype=jnp.int32) + rep2
                for g0 in range(0, NGRP, GG):
                    gs = range(g0, g0 + GG)
                    spks = [plsc.load_gather(spk_v[bi], [pbase + 8 * g])
                            for g in gs]
                    pks = [plsc.load_gather(
                               sct_v,
                               [jnp.bitwise_or(lax.shift_left(s, 1), jpat)])
                           for s in spks]
                    pps = [pos_v[bi][p, pl.ds(L * g, L)] for g in gs]
                    for g, pk, pp in zip(gs, pks, pps):
                        lo = plsc.bitcast(lax.shift_left(pk, 16), jnp.float32) \
                            + plsc.bitcast(lax.shift_left(pp, 16), jnp.float32)
                        hi = plsc.bitcast(jnp.bitwise_and(pk, himask), jnp.float32) \
                            + plsc.bitcast(jnp.bitwise_and(pp, himask), jnp.float32)
                        ref_g = out_v[bi].at[pl.ds(p, 1), pl.ds(32 * g, 32)]
                        plsc.store_scatter(ref_g, [zero16, cole], lo)
                        plsc.store_scatter(ref_g, [zero16, colo], hi)

        for c_ in in_copy(0, 0):
            c_.start()
        for c_ in in_copy(0, 0):
            c_.wait()
        gather_copy(0).start()
        for c_ in in_copy(1, 1):
            c_.start()

        def process(ci, bi):
            nb = 1 - bi
            gather_copy(bi).wait()

            @pl.when(ci >= 2)
            def _():
                out_copy(ci - 2, bi).wait()

            compute(bi)
            out_copy(ci, bi).start()

            @pl.when(ci + 1 < NCH)
            def _():
                for c_ in in_copy(ci + 1, nb):
                    c_.wait()

                gather_copy(nb).start()

                @pl.when(ci + 2 < NCH)
                def _():
                    for c_ in in_copy(ci + 2, bi):
                        c_.start()

        def loop_body(cj, carry):
            for b_ in range(2):
                process(2 * cj + b_, b_)
            return carry

        lax.fori_loop(0, NCH // 2, loop_body, 0)
        out_copy(NCH - 2, 0).wait()
        out_copy(NCH - 1, 1).wait()

    return sc_kernel


_SC_KERNEL = _make_sc_kernel()


def kernel(spikes, space_attn_mask, time_attn_mask, spacestamps, timestamps,
           embed_table, space_pos_table, ln_g, ln_b):
    lnp, sct = _prep_tables(space_pos_table, ln_g, ln_b, embed_table)
    spikes_flat = spikes.reshape(P * C)
    st = spacestamps.reshape(P)
    sct_packed = _pack_bf16_pairs(sct).reshape(MAX_SPIKES * 2)
    lnp_packed = _pack_bf16_pairs(lnp)
    x = _SC_KERNEL(spikes_flat, st, lnp_packed, sct_packed)
    x = x.reshape(B, T, HIDDEN)
    return (x, space_attn_mask, time_attn_mask, time_attn_mask, timestamps)


# speedup vs baseline: 1.6064x; 1.6064x over previous
import functools

import numpy as np
import jax
import jax.numpy as jnp
from jax import lax
from jax.experimental import pallas as pl
from jax.experimental.pallas import tpu as pltpu
from jax.experimental.pallas import tpu_sc as plsc

B = 64
T = 1024
C = 128
MULT = 4
HIDDEN = 512
MAX_SPIKES = 256
N_POS = 1024
SCALE = float(np.sqrt(HIDDEN))
LN_EPS = 1e-5

P = B * T
NC, NS, L = 2, 16, 16
NW = NC * NS
PPW = P // NW
CH = 32
NCH = PPW // CH
NBUF = 4
NGRP = HIDDEN // (2 * L)


def _prep_tables(pos, g, b, emb):
    def body(pos_ref, g_ref, b_ref, emb_ref, lnp_ref, sct_ref):
        x = pos_ref[...]
        mu = jnp.mean(x, axis=-1, keepdims=True)
        var = jnp.mean(jnp.square(x - mu), axis=-1, keepdims=True)
        lnp_ref[...] = (x - mu) / jnp.sqrt(var + LN_EPS) * g_ref[...] + b_ref[...]
        sct_ref[...] = emb_ref[...] * SCALE
    return pl.pallas_call(
        body,
        out_shape=(
            jax.ShapeDtypeStruct((N_POS, HIDDEN), jnp.float32),
            jax.ShapeDtypeStruct((MAX_SPIKES, MULT), jnp.float32),
        ),
    )(pos, g.reshape(1, HIDDEN), b.reshape(1, HIDDEN), emb)


def _pack_bf16_pairs(sct):
    bits = lax.bitcast_convert_type(sct.astype(jnp.bfloat16), jnp.uint16)
    bits = bits.astype(jnp.uint32).reshape(MAX_SPIKES, 2, 2)
    packed = bits[:, :, 0] | (bits[:, :, 1] << 16)
    return lax.bitcast_convert_type(packed, jnp.int32).reshape(MAX_SPIKES * 2)


def _make_sc_kernel():
    mesh = plsc.VectorSubcoreMesh(core_axis_name="c", subcore_axis_name="s")

    @functools.partial(
        pl.kernel,
        mesh=mesh,
        out_type=jax.ShapeDtypeStruct((P, HIDDEN), jnp.float32),
        compiler_params=pltpu.CompilerParams(needs_layout_passes=False),
        scratch_types=(
            [pltpu.VMEM((MAX_SPIKES * 2,), jnp.int32)]
            + [pltpu.VMEM((CH * C,), jnp.int32)] * NBUF
            + [pltpu.VMEM((CH,), jnp.int32)] * NBUF
            + [pltpu.VMEM((CH, HIDDEN), jnp.float32)] * NBUF
            + [pltpu.SemaphoreType.DMA] * (3 * NBUF)
        ),
    )
    def sc_kernel(spk_hbm, st_hbm, lnp_hbm, sct_hbm, out_hbm, sct_v, *rest):
        spk_v = rest[0:NBUF]
        st_v = rest[NBUF:2 * NBUF]
        pos_v = rest[2 * NBUF:3 * NBUF]
        sem_i = rest[3 * NBUF:4 * NBUF]
        sem_g = rest[4 * NBUF:5 * NBUF]
        sem_o = rest[5 * NBUF:6 * NBUF]

        wid = lax.axis_index("s") * NC + lax.axis_index("c")
        base0 = wid * PPW

        pltpu.sync_copy(sct_hbm, sct_v)

        lanes = lax.iota(jnp.int32, L)
        rep2 = lax.shift_right_logical(lanes, 1)
        jpat = jnp.bitwise_and(lanes, 1)
        cole = lax.shift_left(lanes, 1)
        colo = jnp.bitwise_or(cole, 1)
        himask = jnp.full((L,), -65536, dtype=jnp.int32)

        def in_copy(ci, bi):
            base = base0 + ci * CH
            return (
                pltpu.make_async_copy(
                    spk_hbm.at[pl.ds(base * C, CH * C)], spk_v[bi], sem_i[bi]),
                pltpu.make_async_copy(
                    st_hbm.at[pl.ds(base, CH)], st_v[bi], sem_i[bi]),
            )

        def gather_copy(bi):
            return pltpu.make_async_copy(lnp_hbm.at[st_v[bi]], pos_v[bi], sem_g[bi])

        def out_copy(ci, bi):
            base = base0 + ci * CH
            return pltpu.make_async_copy(
                pos_v[bi], out_hbm.at[pl.ds(base, CH)], sem_o[bi])

        def compute(bi):
            GG = 2

            @plsc.parallel_loop(0, CH, unroll=2)
            def pair_body(p):
                zero16 = jnp.bitwise_and(lanes, 0)
                pbase = jnp.full((L,), p * C, dtype=jnp.int32) + rep2
                for g0 in range(0, NGRP, GG):
                    gs = range(g0, g0 + GG)
                    spks = [plsc.load_gather(spk_v[bi], [pbase + 8 * g])
                            for g in gs]
                    pks = [plsc.load_gather(
                               sct_v,
                               [jnp.bitwise_or(lax.shift_left(s, 1), jpat)])
                           for s in spks]
                    for g, pk in zip(gs, pks):
                        lo = plsc.bitcast(lax.shift_left(pk, 16), jnp.float32)
                        hi = plsc.bitcast(jnp.bitwise_and(pk, himask),
                                          jnp.float32)
                        ref_g = pos_v[bi].at[pl.ds(p, 1), pl.ds(32 * g, 32)]
                        plsc.addupdate_scatter(ref_g, [zero16, cole], lo)
                        plsc.addupdate_scatter(ref_g, [zero16, colo], hi)

        for c_ in in_copy(0, 0):
            c_.start()
        for c_ in in_copy(0, 0):
            c_.wait()
        gather_copy(0).start()
        for c_ in in_copy(1, 1):
            c_.start()

        def process(ci, k):
            b1 = (k + 1) % NBUF
            b2 = (k + 2) % NBUF

            @pl.when(ci + 1 < NCH)
            def _():
                for c_ in in_copy(ci + 1, b1):
                    c_.wait()

                @pl.when(ci >= NBUF - 1)
                def _():
                    out_copy(ci - (NBUF - 1), b1).wait()

                gather_copy(b1).start()

                @pl.when(ci + 2 < NCH)
                def _():
                    for c_ in in_copy(ci + 2, b2):
                        c_.start()

            gather_copy(k).wait()
            compute(k)
            out_copy(ci, k).start()

        def loop_body(cj, carry):
            for k in range(NBUF):
                process(NBUF * cj + k, k)
            return carry

        lax.fori_loop(0, NCH // NBUF, loop_body, 0)
        for tail in range(NBUF):
            ci = NCH - NBUF + tail
            out_copy(ci, ci % NBUF).wait()

    return sc_kernel


_SC_KERNEL = _make_sc_kernel()


def kernel(spikes, space_attn_mask, time_attn_mask, spacestamps, timestamps,
           embed_table, space_pos_table, ln_g, ln_b):
    lnp, sct = _prep_tables(space_pos_table, ln_g, ln_b, embed_table)
    spikes_flat = spikes.reshape(P * C)
    st = spacestamps.reshape(P)
    sct_packed = _pack_bf16_pairs(sct)
    x = _SC_KERNEL(spikes_flat, st, lnp, sct_packed)
    x = x.reshape(B, T, HIDDEN)
    return (x, space_attn_mask, time_attn_mask, time_attn_mask, timestamps)
